# Initial kernel scaffold; baseline (speedup 1.0000x reference)
import jax
import jax.numpy as jnp
from jax.experimental import pallas as pl


def kernel(x, W, bias):
    raise NotImplementedError("write your pallas kernel here")


# speedup vs baseline: 1.2653x; 1.2653x over previous
import functools

import jax
import jax.numpy as jnp
from jax.experimental import pallas as pl
from jax.experimental.pallas import tpu as pltpu

D_MODEL = 4096
N_EXPERTS = 64
TOPK = 8
N_GROUPS = 8
GROUP_SIZE = N_EXPERTS // N_GROUPS
TOPK_GROUPS = 4
ROUTE_SCALE = 2.5

BLOCK_T = 512


def _gate_body(x_ref, w_ref, b_ref, wout_ref, iout_ref):
    logits = jnp.dot(x_ref[:], w_ref[:], preferred_element_type=jnp.float32)
    orig = jax.nn.sigmoid(logits)
    scores = orig + b_ref[:]
    bT = scores.shape[0]

    lane8 = jax.lax.broadcasted_iota(jnp.int32, (bT, GROUP_SIZE), 1)
    gsums = []
    for g in range(N_GROUPS):
        sl = scores[:, g * GROUP_SIZE:(g + 1) * GROUP_SIZE]
        m1 = jnp.max(sl, axis=-1, keepdims=True)
        i1 = jnp.min(jnp.where(sl == m1, lane8, GROUP_SIZE), axis=-1, keepdims=True)
        m2 = jnp.max(jnp.where(lane8 == i1, -jnp.inf, sl), axis=-1, keepdims=True)
        gsums.append(m1 + m2)
    gscore = jnp.concatenate(gsums, axis=-1)

    laneg = jax.lax.broadcasted_iota(jnp.int32, (bT, N_GROUPS), 1)
    gmask = jnp.zeros((bT, N_GROUPS), dtype=jnp.float32)
    gwork = gscore
    for _ in range(TOPK_GROUPS):
        m = jnp.max(gwork, axis=-1, keepdims=True)
        gi = jnp.min(jnp.where(gwork == m, laneg, N_GROUPS), axis=-1, keepdims=True)
        hit = laneg == gi
        gmask = jnp.where(hit, 1.0, gmask)
        gwork = jnp.where(hit, -jnp.inf, gwork)

    lane64 = jax.lax.broadcasted_iota(jnp.int32, (bT, N_EXPERTS), 1)
    gid = lane64 // GROUP_SIZE
    lane_mask = jnp.zeros((bT, N_EXPERTS), dtype=jnp.float32)
    for g in range(N_GROUPS):
        lane_mask = jnp.where(gid == g, gmask[:, g:g + 1], lane_mask)
    masked = scores * lane_mask

    widx = []
    wval = []
    work = masked
    for _ in range(TOPK):
        m = jnp.max(work, axis=-1, keepdims=True)
        ei = jnp.min(jnp.where(work == m, lane64, N_EXPERTS), axis=-1, keepdims=True)
        hit = lane64 == ei
        widx.append(ei)
        wval.append(jnp.max(jnp.where(hit, orig, -jnp.inf), axis=-1, keepdims=True))
        work = jnp.where(hit, -jnp.inf, work)
    indices = jnp.concatenate(widx, axis=-1)
    weights = jnp.concatenate(wval, axis=-1)
    weights = weights / jnp.sum(weights, axis=-1, keepdims=True) * ROUTE_SCALE

    wout_ref[:] = weights
    iout_ref[:] = indices


@functools.partial(jax.jit, static_argnames=("interpret",))
def _gate(xf, W, bias2d, interpret=False):
    T = xf.shape[0]
    grid = (T // BLOCK_T,)
    return pl.pallas_call(
        _gate_body,
        grid=grid,
        in_specs=[
            pl.BlockSpec((BLOCK_T, D_MODEL), lambda i: (i, 0)),
            pl.BlockSpec((D_MODEL, N_EXPERTS), lambda i: (0, 0)),
            pl.BlockSpec((1, N_EXPERTS), lambda i: (0, 0)),
        ],
        out_specs=[
            pl.BlockSpec((BLOCK_T, TOPK), lambda i: (i, 0)),
            pl.BlockSpec((BLOCK_T, TOPK), lambda i: (i, 0)),
        ],
        out_shape=[
            jax.ShapeDtypeStruct((T, TOPK), jnp.float32),
            jax.ShapeDtypeStruct((T, TOPK), jnp.int32),
        ],
        compiler_params=pltpu.CompilerParams(
            dimension_semantics=("arbitrary",),
        ),
        interpret=interpret,
    )(xf, W, bias2d)


def kernel(x, W, bias):
    bsz, seq_len, h = x.shape
    xf = x.reshape(-1, h)
    weights, indices = _gate(xf, W, bias.reshape(1, N_EXPERTS))
    return weights.astype(x.dtype), indices


# speedup vs baseline: 4.3244x; 3.4177x over previous
import functools

import jax
import jax.numpy as jnp
from jax.experimental import pallas as pl
from jax.experimental.pallas import tpu as pltpu

D_MODEL = 4096
N_EXPERTS = 64
TOPK = 8
N_GROUPS = 8
GROUP_SIZE = N_EXPERTS // N_GROUPS
TOPK_GROUPS = 4
ROUTE_SCALE = 2.5

BLOCK_T = 512
NEG = -jnp.inf


def _gate_body(x_ref, w_ref, b_ref, wout_ref, iout_ref):
    logits = jnp.dot(x_ref[:], w_ref[:], preferred_element_type=jnp.float32)
    lt = logits.T
    orig = jax.nn.sigmoid(lt)
    scores = orig + b_ref[:]
    bT = scores.shape[1]

    sub8 = jax.lax.broadcasted_iota(jnp.int32, (GROUP_SIZE, bT), 0)

    gsums = []
    for g in range(N_GROUPS):
        sg = scores[g * GROUP_SIZE:(g + 1) * GROUP_SIZE, :]
        m1 = jnp.max(sg, axis=0, keepdims=True)
        i1 = jnp.min(jnp.where(sg == m1, sub8, GROUP_SIZE), axis=0, keepdims=True)
        m2 = jnp.max(jnp.where(sub8 == i1, NEG, sg), axis=0, keepdims=True)
        gsums.append(m1 + m2)
    gscore = jnp.concatenate(gsums, axis=0)

    gmask = []
    gwork = gscore
    for _ in range(TOPK_GROUPS):
        m = jnp.max(gwork, axis=0, keepdims=True)
        gi = jnp.min(jnp.where(gwork == m, sub8, N_GROUPS), axis=0, keepdims=True)
        hit = sub8 == gi
        gmask.append(gi)
        gwork = jnp.where(hit, NEG, gwork)

    keep = jnp.zeros((N_GROUPS, bT), dtype=jnp.float32)
    for gi in gmask:
        keep = jnp.where(sub8 == gi, 1.0, keep)
    pieces = [
        scores[g * GROUP_SIZE:(g + 1) * GROUP_SIZE, :] * keep[g:g + 1, :]
        for g in range(N_GROUPS)
    ]
    masked = jnp.concatenate(pieces, axis=0)

    sub64 = jax.lax.broadcasted_iota(jnp.int32, (N_EXPERTS, bT), 0)
    idxs = []
    wvals = []
    work = masked
    for _ in range(TOPK):
        m = jnp.max(work, axis=0, keepdims=True)
        ei = jnp.min(jnp.where(work == m, sub64, N_EXPERTS), axis=0, keepdims=True)
        hit = sub64 == ei
        idxs.append(ei)
        wvals.append(jnp.max(jnp.where(hit, orig, NEG), axis=0, keepdims=True))
        work = jnp.where(hit, NEG, work)
    indices = jnp.concatenate(idxs, axis=0)
    weights = jnp.concatenate(wvals, axis=0)
    weights = weights / jnp.sum(weights, axis=0, keepdims=True) * ROUTE_SCALE

    wout_ref[:] = weights.T
    iout_ref[:] = indices.T


@functools.partial(jax.jit, static_argnames=("interpret",))
def _gate(xf, W, bias_col, interpret=False):
    T = xf.shape[0]
    grid = (T // BLOCK_T,)
    return pl.pallas_call(
        _gate_body,
        grid=grid,
        in_specs=[
            pl.BlockSpec((BLOCK_T, D_MODEL), lambda i: (i, 0)),
            pl.BlockSpec((D_MODEL, N_EXPERTS), lambda i: (0, 0)),
            pl.BlockSpec((N_EXPERTS, 1), lambda i: (0, 0)),
        ],
        out_specs=[
            pl.BlockSpec((BLOCK_T, TOPK), lambda i: (i, 0)),
            pl.BlockSpec((BLOCK_T, TOPK), lambda i: (i, 0)),
        ],
        out_shape=[
            jax.ShapeDtypeStruct((T, TOPK), jnp.float32),
            jax.ShapeDtypeStruct((T, TOPK), jnp.int32),
        ],
        compiler_params=pltpu.CompilerParams(
            dimension_semantics=("arbitrary",),
        ),
        interpret=interpret,
    )(xf, W, bias_col)


def kernel(x, W, bias):
    bsz, seq_len, h = x.shape
    xf = x.reshape(-1, h)
    weights, indices = _gate(xf, W, bias.reshape(N_EXPERTS, 1))
    return weights.astype(x.dtype), indices


# speedup vs baseline: 4.6474x; 1.0747x over previous
import functools

import jax
import jax.numpy as jnp
from jax.experimental import pallas as pl
from jax.experimental.pallas import tpu as pltpu

D_MODEL = 4096
N_EXPERTS = 64
TOPK = 8
N_GROUPS = 8
GROUP_SIZE = N_EXPERTS // N_GROUPS
TOPK_GROUPS = 4
ROUTE_SCALE = 2.5

BLOCK_T = 1024
NEG = -jnp.inf


def _gate_body(x_ref, w_ref, b_ref, wout_ref, iout_ref):
    logits = jnp.dot(x_ref[:], w_ref[:], preferred_element_type=jnp.float32)
    lt = logits.T
    orig = jax.nn.sigmoid(lt)
    scores = orig + b_ref[:]
    bT = scores.shape[1]

    sub8 = jax.lax.broadcasted_iota(jnp.int32, (GROUP_SIZE, bT), 0)

    gsums = []
    for g in range(N_GROUPS):
        sg = scores[g * GROUP_SIZE:(g + 1) * GROUP_SIZE, :]
        m1 = jnp.max(sg, axis=0, keepdims=True)
        i1 = jnp.min(jnp.where(sg == m1, sub8, GROUP_SIZE), axis=0, keepdims=True)
        m2 = jnp.max(jnp.where(sub8 == i1, NEG, sg), axis=0, keepdims=True)
        gsums.append(m1 + m2)
    gscore = jnp.concatenate(gsums, axis=0)

    gmask = []
    gwork = gscore
    for _ in range(TOPK_GROUPS):
        m = jnp.max(gwork, axis=0, keepdims=True)
        gi = jnp.min(jnp.where(gwork == m, sub8, N_GROUPS), axis=0, keepdims=True)
        hit = sub8 == gi
        gmask.append(gi)
        gwork = jnp.where(hit, NEG, gwork)

    keep = jnp.zeros((N_GROUPS, bT), dtype=jnp.float32)
    for gi in gmask:
        keep = jnp.where(sub8 == gi, 1.0, keep)
    pieces = [
        scores[g * GROUP_SIZE:(g + 1) * GROUP_SIZE, :] * keep[g:g + 1, :]
        for g in range(N_GROUPS)
    ]
    masked = jnp.concatenate(pieces, axis=0)

    sub64 = jax.lax.broadcasted_iota(jnp.int32, (N_EXPERTS, bT), 0)
    idxs = []
    wvals = []
    work = masked
    for _ in range(TOPK):
        m = jnp.max(work, axis=0, keepdims=True)
        ei = jnp.min(jnp.where(work == m, sub64, N_EXPERTS), axis=0, keepdims=True)
        hit = sub64 == ei
        idxs.append(ei)
        wvals.append(jnp.max(jnp.where(hit, orig, NEG), axis=0, keepdims=True))
        work = jnp.where(hit, NEG, work)
    indices = jnp.concatenate(idxs, axis=0)
    weights = jnp.concatenate(wvals, axis=0)
    weights = weights / jnp.sum(weights, axis=0, keepdims=True) * ROUTE_SCALE

    wout_ref[:] = weights.T
    iout_ref[:] = indices.T


@functools.partial(jax.jit, static_argnames=("interpret",))
def _gate(xf, W, bias_col, interpret=False):
    T = xf.shape[0]
    grid = (T // BLOCK_T,)
    return pl.pallas_call(
        _gate_body,
        grid=grid,
        in_specs=[
            pl.BlockSpec((BLOCK_T, D_MODEL), lambda i: (i, 0)),
            pl.BlockSpec((D_MODEL, N_EXPERTS), lambda i: (0, 0)),
            pl.BlockSpec((N_EXPERTS, 1), lambda i: (0, 0)),
        ],
        out_specs=[
            pl.BlockSpec((BLOCK_T, TOPK), lambda i: (i, 0)),
            pl.BlockSpec((BLOCK_T, TOPK), lambda i: (i, 0)),
        ],
        out_shape=[
            jax.ShapeDtypeStruct((T, TOPK), jnp.float32),
            jax.ShapeDtypeStruct((T, TOPK), jnp.int32),
        ],
        compiler_params=pltpu.CompilerParams(
            dimension_semantics=("arbitrary",),
        ),
        interpret=interpret,
    )(xf, W, bias_col)


def kernel(x, W, bias):
    bsz, seq_len, h = x.shape
    xf = x.reshape(-1, h)
    weights, indices = _gate(xf, W, bias.reshape(N_EXPERTS, 1))
    return weights.astype(x.dtype), indices


# speedup vs baseline: 4.6508x; 1.0007x over previous
import functools

import jax
import jax.numpy as jnp
from jax.experimental import pallas as pl
from jax.experimental.pallas import tpu as pltpu

D_MODEL = 4096
N_EXPERTS = 64
TOPK = 8
N_GROUPS = 8
GROUP_SIZE = N_EXPERTS // N_GROUPS
TOPK_GROUPS = 4
ROUTE_SCALE = 2.5

BLOCK_T = 1024
NEG = -jnp.inf


def _gate_body(x1_ref, x2_ref, w_ref, b_ref, wout_ref, iout_ref):
    logits = (
        jnp.dot(x1_ref[:], w_ref[:D_MODEL // 2, :], preferred_element_type=jnp.float32)
        + jnp.dot(x2_ref[:], w_ref[D_MODEL // 2:, :], preferred_element_type=jnp.float32)
    )
    lt = logits.T
    orig = jax.nn.sigmoid(lt)
    scores = orig + b_ref[:]
    bT = scores.shape[1]

    sub8 = jax.lax.broadcasted_iota(jnp.int32, (GROUP_SIZE, bT), 0)

    gsums = []
    for g in range(N_GROUPS):
        sg = scores[g * GROUP_SIZE:(g + 1) * GROUP_SIZE, :]
        m1 = jnp.max(sg, axis=0, keepdims=True)
        i1 = jnp.min(jnp.where(sg == m1, sub8, GROUP_SIZE), axis=0, keepdims=True)
        m2 = jnp.max(jnp.where(sub8 == i1, NEG, sg), axis=0, keepdims=True)
        gsums.append(m1 + m2)
    gscore = jnp.concatenate(gsums, axis=0)

    gmask = []
    gwork = gscore
    for _ in range(TOPK_GROUPS):
        m = jnp.max(gwork, axis=0, keepdims=True)
        gi = jnp.min(jnp.where(gwork == m, sub8, N_GROUPS), axis=0, keepdims=True)
        hit = sub8 == gi
        gmask.append(gi)
        gwork = jnp.where(hit, NEG, gwork)

    keep = jnp.zeros((N_GROUPS, bT), dtype=jnp.float32)
    for gi in gmask:
        keep = jnp.where(sub8 == gi, 1.0, keep)
    pieces = [
        scores[g * GROUP_SIZE:(g + 1) * GROUP_SIZE, :] * keep[g:g + 1, :]
        for g in range(N_GROUPS)
    ]
    masked = jnp.concatenate(pieces, axis=0)

    sub64 = jax.lax.broadcasted_iota(jnp.int32, (N_EXPERTS, bT), 0)
    idxs = []
    wvals = []
    work = masked
    for _ in range(TOPK):
        m = jnp.max(work, axis=0, keepdims=True)
        ei = jnp.min(jnp.where(work == m, sub64, N_EXPERTS), axis=0, keepdims=True)
        hit = sub64 == ei
        idxs.append(ei)
        wvals.append(jnp.max(jnp.where(hit, orig, NEG), axis=0, keepdims=True))
        work = jnp.where(hit, NEG, work)
    indices = jnp.concatenate(idxs, axis=0)
    weights = jnp.concatenate(wvals, axis=0)
    weights = weights / jnp.sum(weights, axis=0, keepdims=True) * ROUTE_SCALE

    wout_ref[:] = weights.T
    iout_ref[:] = indices.T


@functools.partial(jax.jit, static_argnames=("interpret",))
def _gate(xf, W, bias_col, interpret=False):
    T = xf.shape[0]
    grid = (T // BLOCK_T,)
    return pl.pallas_call(
        _gate_body,
        grid=grid,
        in_specs=[
            pl.BlockSpec((BLOCK_T, D_MODEL // 2), lambda i: (i, 0)),
            pl.BlockSpec((BLOCK_T, D_MODEL // 2), lambda i: (i, 1)),
            pl.BlockSpec((D_MODEL, N_EXPERTS), lambda i: (0, 0)),
            pl.BlockSpec((N_EXPERTS, 1), lambda i: (0, 0)),
        ],
        out_specs=[
            pl.BlockSpec((BLOCK_T, TOPK), lambda i: (i, 0)),
            pl.BlockSpec((BLOCK_T, TOPK), lambda i: (i, 0)),
        ],
        out_shape=[
            jax.ShapeDtypeStruct((T, TOPK), jnp.float32),
            jax.ShapeDtypeStruct((T, TOPK), jnp.int32),
        ],
        compiler_params=pltpu.CompilerParams(
            dimension_semantics=("arbitrary",),
        ),
        interpret=interpret,
    )(xf, xf, W, bias_col)


def kernel(x, W, bias):
    bsz, seq_len, h = x.shape
    xf = x.reshape(-1, h)
    weights, indices = _gate(xf, W, bias.reshape(N_EXPERTS, 1))
    return weights.astype(x.dtype), indices
